# baseline (speedup 1.0000x reference)
import functools

import jax
import jax.numpy as jnp
from jax import lax
from jax.experimental import pallas as pl
from jax.experimental.pallas import tpu as pltpu
from jax.experimental.pallas import tpu_sc as plsc

_B = 16
_D = 1024
_L = 16
_NC = 2
_RPC = _B // _NC


def _sc_body(hs_ref, w_ref, b_ref, cu_ref, out_ref,
             cu_v, w_v, piece_v, contrib_v, b_v, gath_v, out_v, shared,
             sem, sem_w, sem_b):
    c = lax.axis_index("c")
    s = lax.axis_index("s")

    @pl.when(s < _RPC)
    def _compute():
        cp_w = pltpu.async_copy(w_ref, w_v, sem_w)
        cp_b = pltpu.async_copy(b_ref, b_v, sem_b)
        pltpu.sync_copy(cu_ref, cu_v)
        lane = lax.iota(jnp.int32, _L)
        pltpu.async_copy(hs_ref.at[cu_v.at[c * _RPC + s]], piece_v,
                         sem).wait()
        cp_w.wait()
        def _fma(i, acc):
            return acc + (piece_v[0, pl.ds(i * _L, _L)]
                          * w_v[pl.ds(i * _L, _L)])
        acc = lax.fori_loop(0, _D // _L, _fma,
                            jnp.zeros((_L,), dtype=jnp.float32), unroll=4)
        logit = jnp.sum(acc)
        contrib_v[...] = jnp.where(lane == s, logit, 0.0)
        pltpu.sync_copy(contrib_v, shared.at[pl.ds(s * _L, _L)])
        cp_b.wait()

    plsc.subcore_barrier()

    @pl.when(s == 0)
    def _finalize():
        pltpu.sync_copy(shared, gath_v)
        tot = gath_v[pl.ds(0, _L)]
        for r in range(1, _RPC):
            tot = tot + gath_v[pl.ds(r * _L, _L)]
        logits = tot + b_v[...]
        out_v[...] = 1.0 / (1.0 + jnp.exp(-logits))
        pltpu.sync_copy(out_v.at[pl.ds(0, _RPC)],
                        out_ref.at[pl.ds(c * _RPC, _RPC)])


@functools.partial(
    pl.kernel,
    out_type=jax.ShapeDtypeStruct((_B,), jnp.float32),
    mesh=plsc.VectorSubcoreMesh(core_axis_name="c", subcore_axis_name="s",
                                num_cores=_NC),
    compiler_params=pltpu.CompilerParams(needs_layout_passes=False),
    scratch_types=[
        pltpu.VMEM((_B, 1), jnp.int32),
        pltpu.VMEM((_D,), jnp.float32),
        pltpu.VMEM((1, _D), jnp.float32),
        pltpu.VMEM((_L,), jnp.float32),
        pltpu.VMEM((_L,), jnp.float32),
        pltpu.VMEM((_RPC * _L,), jnp.float32),
        pltpu.VMEM((_L,), jnp.float32),
        pltpu.VMEM_SHARED((_RPC * _L,), jnp.float32),
        pltpu.SemaphoreType.DMA,
        pltpu.SemaphoreType.DMA,
        pltpu.SemaphoreType.DMA,
    ],
)
def _sc_pool(hs_ref, w_ref, b_ref, cu_ref, out_ref, *scratch):
    _sc_body(hs_ref, w_ref, b_ref, cu_ref, out_ref, *scratch)


def kernel(hidden_states, W, b, cu_seqlens):
    w1 = W.reshape(_D)
    bvec = jnp.broadcast_to(b.astype(jnp.float32), (_L,))
    starts2 = cu_seqlens[:-1].reshape(_B, 1)
    return _sc_pool(hidden_states, w1, bvec, starts2)


# speedup vs baseline: 1.0420x; 1.0420x over previous
import functools

import jax
import jax.numpy as jnp
from jax import lax
from jax.experimental import pallas as pl
from jax.experimental.pallas import tpu as pltpu
from jax.experimental.pallas import tpu_sc as plsc

_B = 16
_D = 1024
_L = 16


def _sc_body(hs_ref, w_ref, b_ref, cu_ref, out_ref,
             cu_v, w_v, piece_v, contrib_v, b_v, gath_v, out_v, shared,
             sem, sem_w, sem_b):
    s = lax.axis_index("s")

    def _compute():
        cp_w = pltpu.async_copy(w_ref, w_v, sem_w)
        cp_b = pltpu.async_copy(b_ref, b_v, sem_b)
        pltpu.sync_copy(cu_ref, cu_v)
        lane = lax.iota(jnp.int32, _L)
        pltpu.async_copy(hs_ref.at[cu_v.at[s]], piece_v, sem).wait()
        cp_w.wait()
        def _fma(i, acc):
            return acc + (piece_v[0, pl.ds(i * _L, _L)]
                          * w_v[pl.ds(i * _L, _L)])
        acc = lax.fori_loop(0, _D // _L, _fma,
                            jnp.zeros((_L,), dtype=jnp.float32), unroll=4)
        logit = jnp.sum(acc)
        contrib_v[...] = jnp.where(lane == s, logit, 0.0)
        pltpu.sync_copy(contrib_v, shared.at[pl.ds(s * _L, _L)])
        cp_b.wait()

    _compute()
    plsc.subcore_barrier()

    @pl.when(s == 0)
    def _finalize():
        pltpu.sync_copy(shared, gath_v)
        tot = gath_v[pl.ds(0, _L)]
        for r in range(1, _B):
            tot = tot + gath_v[pl.ds(r * _L, _L)]
        logits = tot + b_v[...]
        out_v[...] = 1.0 / (1.0 + jnp.exp(-logits))
        pltpu.sync_copy(out_v, out_ref)


@functools.partial(
    pl.kernel,
    out_type=jax.ShapeDtypeStruct((_B,), jnp.float32),
    mesh=plsc.VectorSubcoreMesh(core_axis_name="c", subcore_axis_name="s",
                                num_cores=1),
    compiler_params=pltpu.CompilerParams(needs_layout_passes=False),
    scratch_types=[
        pltpu.VMEM((_B, 1), jnp.int32),
        pltpu.VMEM((_D,), jnp.float32),
        pltpu.VMEM((1, _D), jnp.float32),
        pltpu.VMEM((_L,), jnp.float32),
        pltpu.VMEM((_B,), jnp.float32),
        pltpu.VMEM((_B * _L,), jnp.float32),
        pltpu.VMEM((_B,), jnp.float32),
        pltpu.VMEM_SHARED((_B * _L,), jnp.float32),
        pltpu.SemaphoreType.DMA,
        pltpu.SemaphoreType.DMA,
        pltpu.SemaphoreType.DMA,
    ],
)
def _sc_pool(hs_ref, w_ref, b_ref, cu_ref, out_ref, *scratch):
    _sc_body(hs_ref, w_ref, b_ref, cu_ref, out_ref, *scratch)


def kernel(hidden_states, W, b, cu_seqlens):
    w1 = W.reshape(_D)
    bvec = jnp.broadcast_to(b.astype(jnp.float32), (_B,))
    starts2 = cu_seqlens[:-1].reshape(_B, 1)
    return _sc_pool(hidden_states, w1, bvec, starts2)
